# baseline (speedup 1.0000x reference)
import jax
import jax.numpy as jnp
from jax.experimental import pallas as pl

NUM_CLASSES = 1000
N_ROWS = 16384
SEG = 26
BC = 8
BI = 8192


def _onehot_body(xb_ref, o_ref):
    cls0 = pl.program_id(0) * BC
    cls = cls0 + jax.lax.broadcasted_iota(jnp.int32, (SEG, BC, BI), 1)
    o_ref[...] = (xb_ref[...] == cls).astype(jnp.float32)


def kernel(x):
    xt = x.astype(jnp.int32).T
    xb = jnp.broadcast_to(xt[:, None, :], (SEG, BC, N_ROWS))
    out_t = pl.pallas_call(
        _onehot_body,
        grid=(NUM_CLASSES // BC, N_ROWS // BI),
        in_specs=[pl.BlockSpec((SEG, BC, BI), lambda ci, ii: (0, 0, ii))],
        out_specs=pl.BlockSpec((SEG, BC, BI), lambda ci, ii: (0, ci, ii)),
        out_shape=jax.ShapeDtypeStruct((SEG, NUM_CLASSES, N_ROWS), jnp.float32),
    )(xb)
    return out_t.transpose(2, 0, 1)


# speedup vs baseline: 1.9797x; 1.9797x over previous
import jax
import jax.numpy as jnp
from jax.experimental import pallas as pl

NUM_CLASSES = 1000
N_ROWS = 16384
SEG = 26
BC = 8
BI = 16384


def _onehot_body(xb_ref, o_ref):
    cls0 = pl.program_id(0) * BC
    cls = cls0 + jax.lax.broadcasted_iota(jnp.int32, (SEG, BC, BI), 1)
    o_ref[...] = (xb_ref[...] == cls).astype(jnp.float32)


def kernel(x):
    xt = x.astype(jnp.int32).T
    xb = jnp.broadcast_to(xt[:, None, :], (SEG, BC, N_ROWS))
    out_t = pl.pallas_call(
        _onehot_body,
        grid=(NUM_CLASSES // BC,),
        in_specs=[pl.BlockSpec((SEG, BC, BI), lambda ci: (0, 0, 0))],
        out_specs=pl.BlockSpec((SEG, BC, BI), lambda ci: (0, ci, 0)),
        out_shape=jax.ShapeDtypeStruct((SEG, NUM_CLASSES, N_ROWS), jnp.float32),
    )(xb)
    return out_t.transpose(2, 0, 1)


# speedup vs baseline: 2.0841x; 1.0528x over previous
import jax
import jax.numpy as jnp
from jax.experimental import pallas as pl
from jax.experimental.pallas import tpu as pltpu

NUM_CLASSES = 1000
N_ROWS = 16384
SEG = 26
BC = 8
BI = 16384


def _onehot_body(xt_ref, o_ref, xb_ref):
    ci = pl.program_id(0)

    @pl.when(ci == 0)
    def _():
        xb_ref[...] = jnp.broadcast_to(xt_ref[...][:, None, :], (SEG, BC, BI))

    cls = ci * BC + jax.lax.broadcasted_iota(jnp.int32, (SEG, BC, BI), 1)
    o_ref[...] = (xb_ref[...] == cls).astype(jnp.float32)


def kernel(x):
    xt = x.astype(jnp.int32).T
    out_t = pl.pallas_call(
        _onehot_body,
        grid=(NUM_CLASSES // BC,),
        in_specs=[pl.BlockSpec((SEG, BI), lambda ci: (0, 0))],
        out_specs=pl.BlockSpec((SEG, BC, BI), lambda ci: (0, ci, 0)),
        out_shape=jax.ShapeDtypeStruct((SEG, NUM_CLASSES, N_ROWS), jnp.float32),
        scratch_shapes=[pltpu.VMEM((SEG, BC, BI), jnp.int32)],
    )(xt)
    return out_t.transpose(2, 0, 1)
